# baseline (speedup 1.0000x reference)
import numpy as np
import jax
import jax.numpy as jnp
from jax.experimental import pallas as pl
from jax.experimental.pallas import tpu as pltpu

_C = 192

_SWAPPED = [
    2, 3, 4, 5, 6, 7, 8, 10, 11, 15, 16, 18, 19, 20, 22, 24, 29, 30, 31, 32,
    34, 35, 37, 39, 42, 43, 44, 45, 49, 50, 53, 54, 56, 58, 61, 63, 65, 67,
    69, 70, 72, 77, 78, 80, 81, 82, 83, 85, 90, 92, 94, 96, 99, 101, 102,
    108, 110, 111, 112, 114, 117, 118, 121, 123, 129, 130, 137, 138, 139,
    140, 142, 144, 147, 148, 152, 153, 155, 156, 157, 159, 163, 167, 169,
    173, 174, 175, 176, 177, 178, 179, 183, 184, 185, 186, 188, 189,
]
_MASK = np.zeros(_C, dtype=bool)
_MASK[np.asarray(_SWAPPED)] = True

_G = 8
_P = _C // _G


def _segments(c0):
    segs = []
    for c in range(c0, c0 + _G):
        sw = bool(_MASK[c])
        if segs and segs[-1][2] == sw and segs[-1][1] == c:
            segs[-1] = (segs[-1][0], c + 1, sw)
        else:
            segs.append((c, c + 1, sw))
    return segs


def _body(f1, f2, out5, bufA0, bufA1, bufB0, bufB1, in0, in1, out0, out1):
    bufA = (bufA0, bufA1)
    bufB = (bufB0, bufB1)
    in_sem = (in0, in1)
    out_sem = (out0, out1)

    def in_copies(g):
        par = g % 2
        c0 = g * _G
        return (
            pltpu.make_async_copy(f1.at[:, c0 : c0 + _G], bufA[par], in_sem[par]),
            pltpu.make_async_copy(f2.at[:, c0 : c0 + _G], bufB[par], in_sem[par]),
        )

    def out_copies(g):
        par = g % 2
        c0 = g * _G
        cps = []
        for a, b, sw in _segments(c0):
            j0, j1 = a - c0, b - c0
            h = 1 if sw else 0
            cps.append(
                pltpu.make_async_copy(
                    bufA[par].at[:, j0:j1], out5.at[:, h, a:b], out_sem[par]
                )
            )
            cps.append(
                pltpu.make_async_copy(
                    bufB[par].at[:, j0:j1], out5.at[:, 1 - h, a:b], out_sem[par]
                )
            )
        return cps

    def start(cps):
        for cp in cps:
            cp.start()

    def wait(cps):
        for cp in cps:
            cp.wait()

    start(in_copies(0))
    start(in_copies(1))
    for g in range(_P):
        wait(in_copies(g))
        start(out_copies(g))
        if 1 <= g < _P - 1:
            wait(out_copies(g - 1))
            start(in_copies(g + 1))
    wait(out_copies(_P - 2))
    wait(out_copies(_P - 1))


@jax.jit
def kernel(f1, f2):
    B, C, H, W = f1.shape
    buf = pltpu.VMEM((B, _G, H, W), jnp.float32)
    out = pl.pallas_call(
        _body,
        in_specs=[
            pl.BlockSpec(memory_space=pltpu.MemorySpace.HBM),
            pl.BlockSpec(memory_space=pltpu.MemorySpace.HBM),
        ],
        out_specs=pl.BlockSpec(memory_space=pltpu.MemorySpace.HBM),
        out_shape=jax.ShapeDtypeStruct((B, 2, C, H, W), f1.dtype),
        scratch_shapes=[buf, buf, buf, buf]
        + [pltpu.SemaphoreType.DMA] * 4,
    )(f1, f2)
    return out.reshape(B, 2 * C, H, W)
